# baseline (speedup 1.0000x reference)
import jax
import jax.numpy as jnp
from jax.experimental import pallas as pl
from jax.experimental.pallas import tpu as pltpu

_BM = 256
_NP = 12544
_NT = _NP // 128
_BANDS = _BM // 8
_CH = _BM * _NP


def _body(x_ref, w_ref, flat_hbm, o2d, fbuf, sems):
    i = pl.program_id(0)
    n = pl.num_programs(0)
    slot = i % 2

    o2d[...] = jax.lax.dot_general(
        x_ref[...].astype(jnp.bfloat16),
        w_ref[...],
        dimension_numbers=(((1,), (1,)), ((), ())),
        preferred_element_type=jnp.float32,
    ) + 1.0

    @pl.when(i >= 2)
    def _wait_prev():
        pltpu.make_async_copy(
            fbuf.at[slot], flat_hbm.at[pl.ds((i - 2) * _CH, _CH)], sems.at[slot]
        ).wait()

    fb = fbuf.at[slot]

    def _band(b, carry):
        for c in range(_NT):
            fb[pl.ds((b * _NT + c) * 1024, 1024)] = (
                o2d[pl.ds(8 * b, 8), pl.ds(128 * c, 128)].reshape(1024)
            )
        return carry

    jax.lax.fori_loop(0, _BANDS, _band, 0)

    pltpu.make_async_copy(
        fbuf.at[slot], flat_hbm.at[pl.ds(i * _CH, _CH)], sems.at[slot]
    ).start()

    @pl.when(i == n - 1)
    def _drain():
        for k in (1, 0):
            pltpu.make_async_copy(
                fbuf.at[(i - k) % 2],
                flat_hbm.at[pl.ds((i - k) * _CH, _CH)],
                sems.at[(i - k) % 2],
            ).wait()


def kernel(total_features, norm_weight):
    M, K = total_features.shape
    N = norm_weight.shape[0]
    w_bf = norm_weight.astype(jnp.bfloat16)
    grid = (M // _BM,)
    flat = pl.pallas_call(
        _body,
        grid=grid,
        in_specs=[
            pl.BlockSpec((_BM, K), lambda i: (i, 0)),
            pl.BlockSpec((_NP, K), lambda i: (0, 0)),
        ],
        out_specs=pl.BlockSpec(memory_space=pl.ANY),
        out_shape=jax.ShapeDtypeStruct((M * _NP,), jnp.float32),
        scratch_shapes=[
            pltpu.VMEM((_BM, _NP), jnp.float32),
            pltpu.VMEM((2, _CH), jnp.float32),
            pltpu.SemaphoreType.DMA((2,)),
        ],
        compiler_params=pltpu.CompilerParams(
            dimension_semantics=("arbitrary",),
        ),
    )(total_features, w_bf)
    tiled = flat.reshape(M // 8, _NT, 8, 128).transpose(0, 2, 1, 3)
    return tiled.reshape(M, _NP)[:, :N] - 1.0


# speedup vs baseline: 1.6425x; 1.6425x over previous
import jax
import jax.numpy as jnp
from jax.experimental import pallas as pl
from jax.experimental.pallas import tpu as pltpu


def _mm_body(x_ref, w_ref, o_ref, xbf_ref):
    @pl.when(pl.program_id(0) == 0)
    def _cast_x_once():
        xbf_ref[...] = x_ref[...].astype(jnp.bfloat16)

    o_ref[...] = jax.lax.dot_general(
        xbf_ref[...],
        w_ref[...].astype(jnp.bfloat16),
        dimension_numbers=(((1,), (1,)), ((), ())),
        preferred_element_type=jnp.float32,
    )


def kernel(total_features, norm_weight):
    M, K = total_features.shape
    N = norm_weight.shape[0]
    bn = 1280
    grid = (pl.cdiv(N, bn),)
    return pl.pallas_call(
        _mm_body,
        grid=grid,
        in_specs=[
            pl.BlockSpec((M, K), lambda j: (0, 0)),
            pl.BlockSpec((bn, K), lambda j: (j, 0)),
        ],
        out_specs=pl.BlockSpec((M, bn), lambda j: (0, j)),
        out_shape=jax.ShapeDtypeStruct((M, N), jnp.float32),
        scratch_shapes=[pltpu.VMEM((M, K), jnp.bfloat16)],
        compiler_params=pltpu.CompilerParams(
            dimension_semantics=("arbitrary",),
            vmem_limit_bytes=63 * 1024 * 1024,
        ),
    )(total_features, norm_weight)
